# Initial kernel scaffold; baseline (speedup 1.0000x reference)
import functools

import jax
import jax.numpy as jnp
from jax.experimental import pallas as pl
from jax.experimental.pallas import tpu as pltpu

_K_OUT = 9
_RANKS = 17
_ROWS = 256
_KPAD = 16


def _knn_kernel(x_ref, out_ref, xn_ref, xsq_ref, dist_ref, *, n_total):
    i = pl.program_id(1)

    @pl.when(i == 0)
    def _():
        x = x_ref[0]
        norm = jnp.sqrt(jnp.sum(x * x, axis=0, keepdims=True))
        xn = x / jnp.maximum(norm, 1e-12)
        xn_ref[...] = xn
        xsq_ref[...] = jnp.sum(xn * xn, axis=0, keepdims=True)

    xn_all = xn_ref[...]
    xr = xn_ref[:, pl.ds(i * _ROWS, _ROWS)]

    s = jax.lax.dot_general(
        xr, xn_all,
        dimension_numbers=(((0,), (0,)), ((), ())),
        preferred_element_type=jnp.float32,
    )

    xsq_c = xsq_ref[...]
    xsq_r = jnp.sum(xr * xr, axis=0)[:, None]
    dist_ref[...] = (xsq_r + (-2.0 * s)) + xsq_c

    iota = jax.lax.broadcasted_iota(jnp.int32, (_ROWS, n_total), 1)

    def body(k, _):
        d = dist_ref[...]
        v = jnp.min(d, axis=1, keepdims=True)
        cand = jnp.where(d == v, iota, n_total)
        idx = jnp.min(cand, axis=1, keepdims=True)

        @pl.when(k % 2 == 0)
        def _():
            out_ref[0, :, pl.ds(k // 2, 1)] = idx

        dist_ref[...] = jnp.where(iota == idx, jnp.inf, d)
        return 0

    jax.lax.fori_loop(0, _RANKS, body, 0)


def kernel(x):
    b, c, n, _ = x.shape
    x3 = x[..., 0]

    nn_pad = pl.pallas_call(
        functools.partial(_knn_kernel, n_total=n),
        grid=(b, n // _ROWS),
        in_specs=[pl.BlockSpec((1, c, n), lambda bb, ii: (bb, 0, 0))],
        out_specs=pl.BlockSpec((1, _ROWS, _KPAD), lambda bb, ii: (bb, ii, 0)),
        out_shape=jax.ShapeDtypeStruct((b, n, _KPAD), jnp.int32),
        scratch_shapes=[
            pltpu.VMEM((c, n), jnp.float32),
            pltpu.VMEM((1, n), jnp.float32),
            pltpu.VMEM((_ROWS, n), jnp.float32),
        ],
    )(x3)

    nn_idx = nn_pad[:, :, :_K_OUT]
    center = jnp.broadcast_to(
        jnp.arange(n, dtype=jnp.int32)[None, :, None], (b, n, _K_OUT)
    )
    return jnp.stack((nn_idx, center), axis=0)


# speedup vs baseline: 13.5206x; 13.5206x over previous
import functools

import jax
import jax.numpy as jnp
from jax.experimental import pallas as pl
from jax.experimental.pallas import tpu as pltpu

_K_OUT = 9
_RANKS = 17
_ROWS = 256
_KPAD = 16


def _knn_kernel(x_ref, out_ref, xn_ref, xsq_ref, dist_ref, *, n_total):
    i = pl.program_id(1)

    @pl.when(i == 0)
    def _():
        x = x_ref[0]
        norm = jnp.sqrt(jnp.sum(x * x, axis=0, keepdims=True))
        xn = x / jnp.maximum(norm, 1e-12)
        xn_ref[...] = xn
        xsq_ref[...] = jnp.sum(xn * xn, axis=0, keepdims=True)

    xn_all = xn_ref[...]
    xr = xn_ref[:, pl.ds(i * _ROWS, _ROWS)]

    s = jax.lax.dot_general(
        xr, xn_all,
        dimension_numbers=(((0,), (0,)), ((), ())),
        preferred_element_type=jnp.float32,
    )

    xsq_c = xsq_ref[...]
    xsq_r = jnp.sum(xr * xr, axis=0)[:, None]
    dist_ref[...] = (xsq_r + (-2.0 * s)) + xsq_c

    iota = jax.lax.broadcasted_iota(jnp.int32, (_ROWS, n_total), 1)
    kiota = jax.lax.broadcasted_iota(jnp.int32, (_ROWS, _KPAD), 1)

    def body(k, acc):
        d = dist_ref[...]
        v = jnp.min(d, axis=1, keepdims=True)
        cand = jnp.where(d == v, iota, n_total)
        idx = jnp.min(cand, axis=1, keepdims=True)
        take = jnp.logical_and(k % 2 == 0, kiota == k // 2)
        acc = jnp.where(take, idx, acc)
        dist_ref[...] = jnp.where(iota == idx, jnp.inf, d)
        return acc

    acc0 = jnp.zeros((_ROWS, _KPAD), jnp.int32)
    out_ref[0] = jax.lax.fori_loop(0, _RANKS, body, acc0)


def kernel(x):
    b, c, n, _ = x.shape
    x3 = x[..., 0]

    nn_pad = pl.pallas_call(
        functools.partial(_knn_kernel, n_total=n),
        grid=(b, n // _ROWS),
        in_specs=[pl.BlockSpec((1, c, n), lambda bb, ii: (bb, 0, 0))],
        out_specs=pl.BlockSpec((1, _ROWS, _KPAD), lambda bb, ii: (bb, ii, 0)),
        out_shape=jax.ShapeDtypeStruct((b, n, _KPAD), jnp.int32),
        scratch_shapes=[
            pltpu.VMEM((c, n), jnp.float32),
            pltpu.VMEM((1, n), jnp.float32),
            pltpu.VMEM((_ROWS, n), jnp.float32),
        ],
    )(x3)

    nn_idx = nn_pad[:, :, :_K_OUT]
    center = jnp.broadcast_to(
        jnp.arange(n, dtype=jnp.int32)[None, :, None], (b, n, _K_OUT)
    )
    return jnp.stack((nn_idx, center), axis=0)
